# Initial kernel scaffold; baseline (speedup 1.0000x reference)
import jax
import jax.numpy as jnp
from jax.experimental import pallas as pl


def kernel(u_prev, i_prev, w_user, w_item, u_src, u_dst, i_src, i_dst, edge_index_c0, edge_index_c1):
    raise NotImplementedError("write your pallas kernel here")


# speedup vs baseline: 13.7121x; 13.7121x over previous
import functools

import jax
import jax.numpy as jnp
from jax import lax
from jax.experimental import pallas as pl
from jax.experimental.pallas import tpu as pltpu
from jax.experimental.pallas import tpu_sc as plsc

M = 10000
E = 160000
F = 256
FH = 128
H = 4
CH = 2
BLK = 1024
MP = 10240
NBLK = MP // BLK
CHUNK = 128
NCHUNK = E // CHUNK
NSUB = 16
MAXCH = (NCHUNK + NSUB - 1) // NSUB
SPAD = 40960


def _lrelu(x):
    return jnp.where(x > 0, x, 0.01 * x)


def _tc1_body(up_ref, ip_ref, wu_ref, wi_ref, aus_ref, aud_ref, ais_ref,
              aid_ref, fu_ref, fi_ref, sa_ref, sb_ref):
    up = up_ref[...]
    ip = ip_ref[...]
    wu0 = wu_ref[0]
    wu1 = wu0 + wu_ref[1]
    wi0 = wi_ref[0]
    wi1 = wi0 + wi_ref[1]
    u0 = jnp.dot(up, wu0, preferred_element_type=jnp.float32)
    u1 = jnp.dot(up, wu1, preferred_element_type=jnp.float32)
    i0 = jnp.dot(ip, wi0, preferred_element_type=jnp.float32)
    i1 = jnp.dot(ip, wi1, preferred_element_type=jnp.float32)
    fu_ref[0, 0] = u0[:, :FH]
    fu_ref[0, 1] = u0[:, FH:]
    fu_ref[1, 0] = u1[:, :FH]
    fu_ref[1, 1] = u1[:, FH:]
    fi_ref[0, 0] = i0[:, :FH]
    fi_ref[0, 1] = i0[:, FH:]
    fi_ref[1, 0] = i1[:, :FH]
    fi_ref[1, 1] = i1[:, FH:]

    def sc(a, x):
        return lax.dot_general(a, x, (((1,), (1,)), ((), ())),
                               preferred_element_type=jnp.float32)

    aus0 = aus_ref[:, 0, :]
    aus1 = aus0 + aus_ref[:, 1, :]
    aud0 = aud_ref[:, 0, :]
    aud1 = aud0 + aud_ref[:, 1, :]
    ais0 = ais_ref[:, 0, :]
    ais1 = ais0 + ais_ref[:, 1, :]
    aid0 = aid_ref[:, 0, :]
    aid1 = aid0 + aid_ref[:, 1, :]
    sa_ref[0, 0] = sc(aus0, u0)
    sa_ref[0, 1] = sc(aus1, u1)
    sa_ref[1, 0] = sc(ais0, i0)
    sa_ref[1, 1] = sc(ais1, i1)
    sb_ref[0, 0] = sc(aud0, i0)
    sb_ref[0, 1] = sc(aud1, i1)
    sb_ref[1, 0] = sc(aid0, u0)
    sb_ref[1, 1] = sc(aid1, u1)


def _tc1(u_prev, i_prev, w_user, w_item, aus, aud, ais, aid):
    blk_in = pl.BlockSpec((BLK, F), lambda r: (r, 0))
    full = lambda s: pl.BlockSpec(s, lambda r: tuple(0 for _ in s))
    return pl.pallas_call(
        _tc1_body,
        grid=(NBLK,),
        in_specs=[
            blk_in, blk_in,
            full((CH, F, F)), full((CH, F, F)),
            full((H, CH, F)), full((H, CH, F)), full((H, CH, F)),
            full((H, CH, F)),
        ],
        out_specs=[
            pl.BlockSpec((CH, 2, BLK, FH), lambda r: (0, 0, r, 0)),
            pl.BlockSpec((CH, 2, BLK, FH), lambda r: (0, 0, r, 0)),
            pl.BlockSpec((2, CH, H, BLK), lambda r: (0, 0, 0, r)),
            pl.BlockSpec((2, CH, H, BLK), lambda r: (0, 0, 0, r)),
        ],
        out_shape=[
            jax.ShapeDtypeStruct((CH, 2, M, FH), jnp.float32),
            jax.ShapeDtypeStruct((CH, 2, M, FH), jnp.float32),
            jax.ShapeDtypeStruct((2, CH, H, MP), jnp.float32),
            jax.ShapeDtypeStruct((2, CH, H, MP), jnp.float32),
        ],
    )(u_prev, i_prev, w_user, w_item, aus, aud, ais, aid)


def _tc2_body(sa_ref, sb_ref, pack_ref):
    a = sa_ref[...]
    b = sb_ref[...]
    col = lax.broadcasted_iota(jnp.int32, (2, CH, H, MP), 3)
    bm = jnp.where(col < M, b, -jnp.inf)
    mx = jnp.max(bm, axis=3)
    sh = _lrelu(a + mx[:, :, :, None])
    pack_ref[:, :, 0] = a
    pack_ref[:, :, 1] = sh


def _tc2(sa, sb):
    return pl.pallas_call(
        _tc2_body,
        out_shape=jax.ShapeDtypeStruct((2, CH, 2, H, MP), jnp.float32),
    )(sa, sb)


def _sc_body(fu, fi, sp, db, er0, er1, uo, io, ps, sh_hbm,
             eidx, pidx, sjidx, fidx, pgat, pchunk, featb, wbuf,
             zbuf, s_sh, acc, sem):
    half = lax.axis_index("c")
    tid = lax.axis_index("s")
    lane = lax.iota(jnp.int32, 16)
    lo = (tid * NCHUNK) // NSUB
    hi = ((tid + 1) * NCHUNK) // NSUB

    def _z(i, _):
        zbuf[pl.ds(i * 16, 16)] = jnp.zeros((16,), jnp.float32)
        return 0
    lax.fori_loop(0, 2560 // 16, _z, 0)

    def zero_featb():
        def _zf(r, _):
            for v in range(FH // 16):
                featb[r, pl.ds(v * 16, 16)] = jnp.zeros((16,), jnp.float32)
            return 0
        lax.fori_loop(0, CHUNK, _zf, 0)

    for d in range(2):
        out_ref = uo if d == 0 else io

        zero_featb()
        glo = (tid * 125) // NSUB
        ghi = ((tid + 1) * 125) // NSUB

        def zacc(k, _):
            pltpu.sync_copy(featb.at[pl.ds(0, 80)], acc.at[pl.ds(k * 80, 80)])
            return 0
        lax.fori_loop(glo, ghi, zacc, 0)
        plsc.subcore_barrier()

        for c in range(CH):
            er = er0 if c == 0 else er1
            segrow = 0 if d == 0 else 1
            gatrow = 1 - segrow
            aoff = ((d * CH + c) * 2 + 0) * H * MP
            shoff = ((d * CH + c) * 2 + 1) * H * MP
            boff = (d * CH + c) * H * MP
            ftab = fi if d == 0 else fu

            pltpu.sync_copy(zbuf, s_sh.at[pl.ds(tid * 2560, 2560)])
            plsc.subcore_barrier()

            def p1(k, _):
                koff = k - lo
                pltpu.sync_copy(er.at[segrow, pl.ds(k * CHUNK, CHUNK)],
                                eidx.at[0])
                pltpu.sync_copy(er.at[gatrow, pl.ds(k * CHUNK, CHUNK)],
                                eidx.at[1])
                for g in range(CHUNK // 16):
                    seg16 = eidx[0, pl.ds(g * 16, 16)]
                    gat16 = eidx[1, pl.ds(g * 16, 16)]
                    for j in range(H):
                        sjidx[j, pl.ds(g * 16, 16)] = seg16 + j * M
                        pidx[j, pl.ds(g * 16, 16)] = seg16 + (aoff + j * MP)
                        pidx[H + j, pl.ds(g * 16, 16)] = (
                            seg16 + (shoff + j * MP))
                        pidx[2 * H + j, pl.ds(g * 16, 16)] = (
                            gat16 + (boff + j * MP))
                hs = [pltpu.async_copy(sp.at[pidx.at[r]], pgat.at[r], sem)
                      for r in range(2 * H)]
                hs += [pltpu.async_copy(db.at[pidx.at[2 * H + j]],
                                        pgat.at[2 * H + j], sem)
                       for j in range(H)]
                for h in hs:
                    h.wait()
                for g in range(CHUNK // 16):
                    for j in range(H):
                        a = pgat[j, pl.ds(g * 16, 16)]
                        sh = pgat[H + j, pl.ds(g * 16, 16)]
                        b = pgat[2 * H + j, pl.ds(g * 16, 16)]
                        p = jnp.exp(_lrelu(a + b) - sh)
                        pchunk[j, pl.ds(g * 16, 16)] = p
                hs = [pltpu.async_copy(pchunk.at[j],
                                       ps.at[half, j, pl.ds(k * CHUNK, CHUNK)],
                                       sem)
                      for j in range(H)]
                for j in range(H):
                    pltpu.sync_copy(pchunk.at[j], s_sh.at[sjidx.at[j]],
                                    add=True)
                for h in hs:
                    h.wait()
                return 0
            lax.fori_loop(lo, hi, p1, 0)
            plsc.subcore_barrier()

            @pl.when(tid == 0)
            def _stage_s():
                pltpu.sync_copy(s_sh, sh_hbm.at[pl.ds(half * SPAD, SPAD)])
            plsc.subcore_barrier()

            def p2(k, _):
                pltpu.sync_copy(er.at[segrow, pl.ds(k * CHUNK, CHUNK)],
                                eidx.at[0])
                pltpu.sync_copy(er.at[gatrow, pl.ds(k * CHUNK, CHUNK)],
                                eidx.at[1])
                foff = (c * 2 + half) * M
                for g in range(CHUNK // 16):
                    seg16 = eidx[0, pl.ds(g * 16, 16)]
                    gat16 = eidx[1, pl.ds(g * 16, 16)]
                    fidx[0, pl.ds(g * 16, 16)] = gat16 + foff
                    for j in range(H):
                        sjidx[j, pl.ds(g * 16, 16)] = (
                            seg16 + (j * M + half * SPAD))
                hs = [pltpu.async_copy(ps.at[half, j, pl.ds(k * CHUNK, CHUNK)],
                                       pchunk.at[j], sem)
                      for j in range(H)]
                hs += [pltpu.async_copy(sh_hbm.at[sjidx.at[j]],
                                        pgat.at[j], sem)
                       for j in range(H)]
                for h in hs:
                    h.wait()
                for g in range(CHUNK // 16):
                    w = jnp.zeros((16,), jnp.float32)
                    for j in range(H):
                        sj = pgat[j, pl.ds(g * 16, 16)]
                        pj = pchunk[j, pl.ds(g * 16, 16)]
                        w = w + pj / sj
                    wbuf[pl.ds(g * 16, 16)] = w
                pltpu.async_copy(ftab.at[fidx.at[0]], featb, sem).wait()

                def mul_row(e, _):
                    ws = plsc.load_gather(
                        wbuf, [jnp.full((16,), e, jnp.int32)])
                    for v in range(FH // 16):
                        featb[e, pl.ds(v * 16, 16)] = (
                            featb[e, pl.ds(v * 16, 16)] * ws)
                    return 0
                lax.fori_loop(0, CHUNK, mul_row, 0)
                pltpu.sync_copy(featb, acc.at[eidx.at[0]], add=True)
                return 0
            lax.fori_loop(lo, hi, p2, 0)
            plsc.subcore_barrier()

        def wout(k, _):
            pltpu.sync_copy(acc.at[pl.ds(k * 80, 80)], featb.at[pl.ds(0, 80)])

            def relu_row(r, _):
                for v in range(FH // 16):
                    x = featb[r, pl.ds(v * 16, 16)]
                    featb[r, pl.ds(v * 16, 16)] = jnp.maximum(x, 0.0)
                return 0
            lax.fori_loop(0, 80, relu_row, 0)
            pltpu.sync_copy(featb.at[pl.ds(0, 80)],
                            out_ref.at[half, pl.ds(k * 80, 80)])
            return 0
        lax.fori_loop(glo, ghi, wout, 0)
        plsc.subcore_barrier()


def _sc_call(fu2, fi2, sp2, db2, e0, e1):
    mesh = plsc.VectorSubcoreMesh(core_axis_name="c", subcore_axis_name="s")
    return pl.kernel(
        _sc_body,
        out_type=[
            jax.ShapeDtypeStruct((2, M, FH), jnp.float32),
            jax.ShapeDtypeStruct((2, M, FH), jnp.float32),
            jax.ShapeDtypeStruct((2, H, E), jnp.float32),
            jax.ShapeDtypeStruct((2 * SPAD,), jnp.float32),
        ],
        mesh=mesh,
        compiler_params=pltpu.CompilerParams(needs_layout_passes=False),
        scratch_types=[
            pltpu.VMEM((2, CHUNK), jnp.int32),
            pltpu.VMEM((3 * H, CHUNK), jnp.int32),
            pltpu.VMEM((H, CHUNK), jnp.int32),
            pltpu.VMEM((1, CHUNK), jnp.int32),
            pltpu.VMEM((3 * H, CHUNK), jnp.float32),
            pltpu.VMEM((H, CHUNK), jnp.float32),
            pltpu.VMEM((CHUNK, FH), jnp.float32),
            pltpu.VMEM((CHUNK,), jnp.float32),
            pltpu.VMEM((2560,), jnp.float32),
            pltpu.VMEM_SHARED((SPAD,), jnp.float32),
            pltpu.VMEM_SHARED((M, FH), jnp.float32),
            pltpu.SemaphoreType.DMA,
        ],
    )(fu2, fi2, sp2, db2, e0, e1)


def kernel(u_prev, i_prev, w_user, w_item, u_src, u_dst, i_src, i_dst,
           edge_index_c0, edge_index_c1):
    e0 = edge_index_c0.astype(jnp.int32)
    e1 = edge_index_c1.astype(jnp.int32)
    fu, fi, sa, sb = _tc1(u_prev, i_prev, w_user, w_item,
                          u_src, u_dst, i_src, i_dst)
    pack = _tc2(sa, sb)
    uo, io, _ps, _sh = _sc_call(fu.reshape(2 * CH * M, FH),
                      fi.reshape(2 * CH * M, FH),
                      pack.reshape(2 * CH * 2 * H * MP),
                      sb.reshape(2 * CH * H * MP),
                      e0, e1)
    u_out = jnp.moveaxis(uo, 0, 1).reshape(M, F)
    i_out = jnp.moveaxis(io, 0, 1).reshape(M, F)
    return (u_out, i_out)


# speedup vs baseline: 15.4570x; 1.1272x over previous
import functools

import jax
import jax.numpy as jnp
from jax import lax
from jax.experimental import pallas as pl
from jax.experimental.pallas import tpu as pltpu
from jax.experimental.pallas import tpu_sc as plsc

M = 10000
E = 160000
F = 256
FH = 128
H = 4
CH = 2
BLK = 1024
MP = 10240
NBLK = MP // BLK
CHUNK = 128
NCHUNK = E // CHUNK
NSUB = 16
MAXCH = (NCHUNK + NSUB - 1) // NSUB
SPAD = 40960


def _lrelu(x):
    return jnp.where(x > 0, x, 0.01 * x)


def _tc1_body(up_ref, ip_ref, wu_ref, wi_ref, aus_ref, aud_ref, ais_ref,
              aid_ref, fu_ref, fi_ref, sa_ref, sb_ref):
    up = up_ref[...]
    ip = ip_ref[...]
    wu0 = wu_ref[0]
    wu1 = wu0 + wu_ref[1]
    wi0 = wi_ref[0]
    wi1 = wi0 + wi_ref[1]
    u0 = jnp.dot(up, wu0, preferred_element_type=jnp.float32)
    u1 = jnp.dot(up, wu1, preferred_element_type=jnp.float32)
    i0 = jnp.dot(ip, wi0, preferred_element_type=jnp.float32)
    i1 = jnp.dot(ip, wi1, preferred_element_type=jnp.float32)
    fu_ref[0, 0] = u0[:, :FH]
    fu_ref[0, 1] = u0[:, FH:]
    fu_ref[1, 0] = u1[:, :FH]
    fu_ref[1, 1] = u1[:, FH:]
    fi_ref[0, 0] = i0[:, :FH]
    fi_ref[0, 1] = i0[:, FH:]
    fi_ref[1, 0] = i1[:, :FH]
    fi_ref[1, 1] = i1[:, FH:]

    def sc(a, x):
        return lax.dot_general(a, x, (((1,), (1,)), ((), ())),
                               preferred_element_type=jnp.float32)

    aus0 = aus_ref[:, 0, :]
    aus1 = aus0 + aus_ref[:, 1, :]
    aud0 = aud_ref[:, 0, :]
    aud1 = aud0 + aud_ref[:, 1, :]
    ais0 = ais_ref[:, 0, :]
    ais1 = ais0 + ais_ref[:, 1, :]
    aid0 = aid_ref[:, 0, :]
    aid1 = aid0 + aid_ref[:, 1, :]
    sa_ref[0, 0] = sc(aus0, u0)
    sa_ref[0, 1] = sc(aus1, u1)
    sa_ref[1, 0] = sc(ais0, i0)
    sa_ref[1, 1] = sc(ais1, i1)
    sb_ref[0, 0] = sc(aud0, i0)
    sb_ref[0, 1] = sc(aud1, i1)
    sb_ref[1, 0] = sc(aid0, u0)
    sb_ref[1, 1] = sc(aid1, u1)


def _tc1(u_prev, i_prev, w_user, w_item, aus, aud, ais, aid):
    blk_in = pl.BlockSpec((BLK, F), lambda r: (r, 0))
    full = lambda s: pl.BlockSpec(s, lambda r: tuple(0 for _ in s))
    return pl.pallas_call(
        _tc1_body,
        grid=(NBLK,),
        in_specs=[
            blk_in, blk_in,
            full((CH, F, F)), full((CH, F, F)),
            full((H, CH, F)), full((H, CH, F)), full((H, CH, F)),
            full((H, CH, F)),
        ],
        out_specs=[
            pl.BlockSpec((CH, 2, BLK, FH), lambda r: (0, 0, r, 0)),
            pl.BlockSpec((CH, 2, BLK, FH), lambda r: (0, 0, r, 0)),
            pl.BlockSpec((2, CH, H, BLK), lambda r: (0, 0, 0, r)),
            pl.BlockSpec((2, CH, H, BLK), lambda r: (0, 0, 0, r)),
        ],
        out_shape=[
            jax.ShapeDtypeStruct((CH, 2, M, FH), jnp.float32),
            jax.ShapeDtypeStruct((CH, 2, M, FH), jnp.float32),
            jax.ShapeDtypeStruct((2, CH, H, MP), jnp.float32),
            jax.ShapeDtypeStruct((2, CH, H, MP), jnp.float32),
        ],
    )(u_prev, i_prev, w_user, w_item, aus, aud, ais, aid)


def _tc2_body(sa_ref, sb_ref, pack_ref):
    a = sa_ref[...]
    b = sb_ref[...]
    col = lax.broadcasted_iota(jnp.int32, (2, CH, H, MP), 3)
    bm = jnp.where(col < M, b, -jnp.inf)
    mx = jnp.max(bm, axis=3)
    sh = _lrelu(a + mx[:, :, :, None])
    pack_ref[:, :, 0] = a
    pack_ref[:, :, 1] = sh


def _tc2(sa, sb):
    return pl.pallas_call(
        _tc2_body,
        out_shape=jax.ShapeDtypeStruct((2, CH, 2, H, MP), jnp.float32),
    )(sa, sb)


def _sc_body(fu, fi, sp, db, er0, er1, uo, io, ps, sh_hbm,
             eidx, pidx, sjidx, fidx, pgat, pchunk, featb, wbuf,
             zbuf, s_sh, acc, sem, semf):
    half = lax.axis_index("c")
    tid = lax.axis_index("s")
    lane = lax.iota(jnp.int32, 16)
    lo = (tid * NCHUNK) // NSUB
    hi = ((tid + 1) * NCHUNK) // NSUB

    def _z(i, _):
        zbuf[pl.ds(i * 16, 16)] = jnp.zeros((16,), jnp.float32)
        return 0
    lax.fori_loop(0, 2560 // 16, _z, 0)

    def zero_featb():
        def _zf(r, _):
            for v in range(FH // 16):
                featb[r, pl.ds(v * 16, 16)] = jnp.zeros((16,), jnp.float32)
            return 0
        lax.fori_loop(0, CHUNK, _zf, 0)

    for d in range(2):
        out_ref = uo if d == 0 else io

        zero_featb()
        glo = (tid * 125) // NSUB
        ghi = ((tid + 1) * 125) // NSUB

        def zacc(k, _):
            pltpu.sync_copy(featb.at[pl.ds(0, 80)], acc.at[pl.ds(k * 80, 80)])
            return 0
        lax.fori_loop(glo, ghi, zacc, 0)
        plsc.subcore_barrier()

        for c in range(CH):
            er = er0 if c == 0 else er1
            segrow = 0 if d == 0 else 1
            gatrow = 1 - segrow
            aoff = ((d * CH + c) * 2 + 0) * H * MP
            shoff = ((d * CH + c) * 2 + 1) * H * MP
            boff = (d * CH + c) * H * MP
            ftab = fi if d == 0 else fu

            pltpu.sync_copy(zbuf, s_sh.at[pl.ds(tid * 2560, 2560)])
            plsc.subcore_barrier()

            def p1(k, _):
                pltpu.sync_copy(er.at[0, pl.ds(k * CHUNK, CHUNK)], eidx.at[0])
                pltpu.sync_copy(er.at[1, pl.ds(k * CHUNK, CHUNK)], eidx.at[1])
                for g in range(CHUNK // 16):
                    seg16 = eidx[segrow, pl.ds(g * 16, 16)]
                    gat16 = eidx[gatrow, pl.ds(g * 16, 16)]
                    for j in range(H):
                        sjidx[j, pl.ds(g * 16, 16)] = seg16 + j * M
                        pidx[j, pl.ds(g * 16, 16)] = seg16 + (aoff + j * MP)
                        pidx[H + j, pl.ds(g * 16, 16)] = (
                            seg16 + (shoff + j * MP))
                        pidx[2 * H + j, pl.ds(g * 16, 16)] = (
                            gat16 + (boff + j * MP))
                hs = [pltpu.async_copy(sp.at[pidx.at[r]], pgat.at[r], sem)
                      for r in range(2 * H)]
                hs += [pltpu.async_copy(db.at[pidx.at[2 * H + j]],
                                        pgat.at[2 * H + j], sem)
                       for j in range(H)]
                for h in hs:
                    h.wait()
                for g in range(CHUNK // 16):
                    for j in range(H):
                        a = pgat[j, pl.ds(g * 16, 16)]
                        sh = pgat[H + j, pl.ds(g * 16, 16)]
                        b = pgat[2 * H + j, pl.ds(g * 16, 16)]
                        p = jnp.exp(_lrelu(a + b) - sh)
                        pchunk[j, pl.ds(g * 16, 16)] = p
                hs = [pltpu.async_copy(pchunk.at[j],
                                       ps.at[half, j, pl.ds(k * CHUNK, CHUNK)],
                                       sem)
                      for j in range(H)]
                for j in range(H):
                    pltpu.sync_copy(pchunk.at[j], s_sh.at[sjidx.at[j]],
                                    add=True)
                for h in hs:
                    h.wait()
                return 0
            lax.fori_loop(lo, hi, p1, 0)
            plsc.subcore_barrier()

            @pl.when(tid == 0)
            def _stage_s():
                pltpu.sync_copy(s_sh, sh_hbm.at[pl.ds(half * SPAD, SPAD)])
            plsc.subcore_barrier()

            def p2(k, _):
                pltpu.sync_copy(er.at[0, pl.ds(k * CHUNK, CHUNK)], eidx.at[0])
                pltpu.sync_copy(er.at[1, pl.ds(k * CHUNK, CHUNK)], eidx.at[1])
                foff = (c * 2 + half) * M
                for g in range(CHUNK // 16):
                    seg16 = eidx[segrow, pl.ds(g * 16, 16)]
                    gat16 = eidx[gatrow, pl.ds(g * 16, 16)]
                    fidx[0, pl.ds(g * 16, 16)] = gat16 + foff
                    for j in range(H):
                        sjidx[j, pl.ds(g * 16, 16)] = (
                            seg16 + (j * M + half * SPAD))
                hf = pltpu.async_copy(ftab.at[fidx.at[0]], featb, semf)
                hs = [pltpu.async_copy(ps.at[half, j, pl.ds(k * CHUNK, CHUNK)],
                                       pchunk.at[j], sem)
                      for j in range(H)]
                hs += [pltpu.async_copy(sh_hbm.at[sjidx.at[j]],
                                        pgat.at[j], sem)
                       for j in range(H)]
                for h in hs:
                    h.wait()
                for g in range(CHUNK // 16):
                    w = jnp.zeros((16,), jnp.float32)
                    for j in range(H):
                        sj = pgat[j, pl.ds(g * 16, 16)]
                        pj = pchunk[j, pl.ds(g * 16, 16)]
                        w = w + pj / sj
                    wbuf[pl.ds(g * 16, 16)] = w
                hf.wait()

                def mul_row(e, _):
                    ws = plsc.load_gather(
                        wbuf, [jnp.full((16,), e, jnp.int32)])
                    for v in range(FH // 16):
                        featb[e, pl.ds(v * 16, 16)] = (
                            featb[e, pl.ds(v * 16, 16)] * ws)
                    return 0
                lax.fori_loop(0, CHUNK, mul_row, 0)
                pltpu.sync_copy(featb, acc.at[eidx.at[segrow]], add=True)
                return 0
            lax.fori_loop(lo, hi, p2, 0)
            plsc.subcore_barrier()

        def wout(k, _):
            pltpu.sync_copy(acc.at[pl.ds(k * 80, 80)], featb.at[pl.ds(0, 80)])

            def relu_row(r, _):
                for v in range(FH // 16):
                    x = featb[r, pl.ds(v * 16, 16)]
                    featb[r, pl.ds(v * 16, 16)] = jnp.maximum(x, 0.0)
                return 0
            lax.fori_loop(0, 80, relu_row, 0)
            pltpu.sync_copy(featb.at[pl.ds(0, 80)],
                            out_ref.at[half, pl.ds(k * 80, 80)])
            return 0
        lax.fori_loop(glo, ghi, wout, 0)
        plsc.subcore_barrier()


def _sc_call(fu2, fi2, sp2, db2, e0, e1):
    mesh = plsc.VectorSubcoreMesh(core_axis_name="c", subcore_axis_name="s")
    return pl.kernel(
        _sc_body,
        out_type=[
            jax.ShapeDtypeStruct((2, M, FH), jnp.float32),
            jax.ShapeDtypeStruct((2, M, FH), jnp.float32),
            jax.ShapeDtypeStruct((2, H, E), jnp.float32),
            jax.ShapeDtypeStruct((2 * SPAD,), jnp.float32),
        ],
        mesh=mesh,
        compiler_params=pltpu.CompilerParams(needs_layout_passes=False),
        scratch_types=[
            pltpu.VMEM((2, CHUNK), jnp.int32),
            pltpu.VMEM((3 * H, CHUNK), jnp.int32),
            pltpu.VMEM((H, CHUNK), jnp.int32),
            pltpu.VMEM((1, CHUNK), jnp.int32),
            pltpu.VMEM((3 * H, CHUNK), jnp.float32),
            pltpu.VMEM((H, CHUNK), jnp.float32),
            pltpu.VMEM((CHUNK, FH), jnp.float32),
            pltpu.VMEM((CHUNK,), jnp.float32),
            pltpu.VMEM((2560,), jnp.float32),
            pltpu.VMEM_SHARED((SPAD,), jnp.float32),
            pltpu.VMEM_SHARED((M, FH), jnp.float32),
            pltpu.SemaphoreType.DMA,
            pltpu.SemaphoreType.DMA,
        ],
    )(fu2, fi2, sp2, db2, e0, e1)


def kernel(u_prev, i_prev, w_user, w_item, u_src, u_dst, i_src, i_dst,
           edge_index_c0, edge_index_c1):
    e0 = edge_index_c0.astype(jnp.int32)
    e1 = edge_index_c1.astype(jnp.int32)
    fu, fi, sa, sb = _tc1(u_prev, i_prev, w_user, w_item,
                          u_src, u_dst, i_src, i_dst)
    pack = _tc2(sa, sb)
    uo, io, _ps, _sh = _sc_call(fu.reshape(2 * CH * M, FH),
                      fi.reshape(2 * CH * M, FH),
                      pack.reshape(2 * CH * 2 * H * MP),
                      sb.reshape(2 * CH * H * MP),
                      e0, e1)
    u_out = jnp.moveaxis(uo, 0, 1).reshape(M, F)
    i_out = jnp.moveaxis(io, 0, 1).reshape(M, F)
    return (u_out, i_out)
